# baseline (speedup 1.0000x reference)
import jax
import jax.numpy as jnp
from jax.experimental import pallas as pl

_N = 8192
_D = 64
_BM = 256
_NSPLIT = 4
_KS = _N // _NSPLIT


def _fused_block(*refs):
    adj_refs = refs[:_NSPLIT]
    haug_ref, hblk_ref, wt_ref, b_ref, out_ref = refs[_NSPLIT:]
    haug = haug_ref[...]
    acc = None
    for j in range(_NSPLIT):
        part = jnp.dot(
            adj_refs[j][...],
            haug[j * _KS : (j + 1) * _KS, :],
            preferred_element_type=jnp.float32,
        )
        acc = part if acc is None else acc + part
    pooled = acc[:, :_D] / acc[:, _D : _D + 1]
    cat = jnp.concatenate([hblk_ref[...], pooled], axis=1)
    out_ref[...] = (
        jnp.dot(cat, wt_ref[...], preferred_element_type=jnp.float32) + b_ref[...]
    )


@jax.jit
def kernel(h, adj, W_lg, b_lg):
    n, d = h.shape
    wt = W_lg.T
    b = b_lg.reshape(1, d)
    ones = jnp.ones((n, 1), jnp.float32)
    zeros = jnp.zeros((n, d - 1), jnp.float32)
    haug = jnp.concatenate([h, ones, zeros], axis=1)
    grid = (n // _BM,)

    def slice_spec(j):
        return pl.BlockSpec((_BM, _KS), lambda i, j=j: (i, j))

    return pl.pallas_call(
        _fused_block,
        grid=grid,
        in_specs=[slice_spec(j) for j in range(_NSPLIT)]
        + [
            pl.BlockSpec((n, 2 * d), lambda i: (0, 0)),
            pl.BlockSpec((_BM, d), lambda i: (i, 0)),
            pl.BlockSpec((2 * d, d), lambda i: (0, 0)),
            pl.BlockSpec((1, d), lambda i: (0, 0)),
        ],
        out_specs=pl.BlockSpec((_BM, d), lambda i: (i, 0)),
        out_shape=jax.ShapeDtypeStruct((n, d), jnp.float32),
    )(*([adj] * _NSPLIT), haug, h, wt, b)


# speedup vs baseline: 1.0338x; 1.0338x over previous
import jax
import jax.numpy as jnp
from jax.experimental import pallas as pl
from jax.experimental.pallas import tpu as pltpu

_N = 8192
_D = 64
_BM = 256
_S = 4
_NB = _N // _BM


def _fused_block(adj_hbm, h_ref, hblk_ref, wt_ref, b_ref, out_ref, buf, sem):
    i = pl.program_id(0)

    def start_copy(block, slot):
        pltpu.make_async_copy(
            adj_hbm.at[pl.ds(block * _BM, _BM), :], buf.at[slot], sem.at[slot]
        ).start()

    @pl.when(i == 0)
    def _prologue():
        for k in range(_S - 1):
            start_copy(k, k)

    nxt = i + _S - 1

    @pl.when(nxt < _NB)
    def _prefetch():
        start_copy(nxt, jax.lax.rem(nxt, _S))

    slot = jax.lax.rem(i, _S)
    pltpu.make_async_copy(
        adj_hbm.at[pl.ds(i * _BM, _BM), :], buf.at[slot], sem.at[slot]
    ).wait()

    adj = buf[slot]
    acc = jnp.dot(adj, h_ref[...], preferred_element_type=jnp.float32)
    rs = jnp.sum(adj, axis=1, keepdims=True)
    pooled = acc / rs
    cat = jnp.concatenate([hblk_ref[...], pooled], axis=1)
    out_ref[...] = (
        jnp.dot(cat, wt_ref[...], preferred_element_type=jnp.float32) + b_ref[...]
    )


@jax.jit
def kernel(h, adj, W_lg, b_lg):
    n, d = h.shape
    wt = W_lg.T
    b = b_lg.reshape(1, d)
    grid = (_NB,)
    return pl.pallas_call(
        _fused_block,
        grid=grid,
        in_specs=[
            pl.BlockSpec(memory_space=pl.ANY),
            pl.BlockSpec((n, d), lambda i: (0, 0)),
            pl.BlockSpec((_BM, d), lambda i: (i, 0)),
            pl.BlockSpec((2 * d, d), lambda i: (0, 0)),
            pl.BlockSpec((1, d), lambda i: (0, 0)),
        ],
        out_specs=pl.BlockSpec((_BM, d), lambda i: (i, 0)),
        out_shape=jax.ShapeDtypeStruct((n, d), jnp.float32),
        scratch_shapes=[
            pltpu.VMEM((_S, _BM, _N), jnp.float32),
            pltpu.SemaphoreType.DMA((_S,)),
        ],
    )(adj, h, h, wt, b)
